# Initial kernel scaffold; baseline (speedup 1.0000x reference)
import jax
import jax.numpy as jnp
from jax.experimental import pallas as pl


def kernel(x, topk_ids, topk_weight, gate_w, up_w, down_w):
    raise NotImplementedError("write your pallas kernel here")


# speedup vs baseline: 2.6571x; 2.6571x over previous
import functools

import jax
import jax.numpy as jnp
from jax import lax
from jax.experimental import pallas as pl
from jax.experimental.pallas import tpu as pltpu
from jax.experimental.pallas import tpu_sc as plsc

E = 8
D = 1024
FF = 2048
T = 2048
BLK = 256
NB = T // BLK
S = NB + E - 1
RBLK = 128
RNB = T // RBLK
SM = 32
NWORKERS = 32
CHUNK = T // NWORKERS


def _dg(a, b, ca, cb):
    return lax.dot_general(a, b, (((ca,), (cb,)), ((), ())),
                           preferred_element_type=jnp.float32)


def _routing_body(ids_ref, pos_ref, meta_ref):
    e_row = lax.broadcasted_iota(jnp.int32, (1, E), 1)
    tri_r = lax.broadcasted_iota(jnp.int32, (RBLK, RBLK), 0)
    tri_c = lax.broadcasted_iota(jnp.int32, (RBLK, RBLK), 1)
    tri = (tri_r >= tri_c).astype(jnp.float32)

    def pass1(c, running):
        ids_c = ids_ref[pl.ds(c * RBLK, RBLK), :]
        oh = (ids_c == e_row).astype(jnp.float32)
        ranks = _dg(tri, oh, 1, 0) + running
        grank = jnp.sum(oh * (ranks - 1.0), axis=1, keepdims=True)
        pos_ref[pl.ds(c * RBLK, RBLK), :] = grank.astype(jnp.int32)
        return ranks[RBLK - 1:RBLK, :]

    counts = lax.fori_loop(0, RNB, pass1, jnp.zeros((1, E), jnp.float32))

    lt8_r = lax.broadcasted_iota(jnp.int32, (E, E), 0)
    lt8_c = lax.broadcasted_iota(jnp.int32, (E, E), 1)
    lt8 = (lt8_r < lt8_c).astype(jnp.float32)
    off_excl = _dg(counts, lt8, 1, 0)
    cum_incl = off_excl + counts

    def pass2(c, carry):
        ids_c = ids_ref[pl.ds(c * RBLK, RBLK), :]
        oh = (ids_c == e_row).astype(jnp.float32)
        add = jnp.sum(oh * off_excl, axis=1, keepdims=True).astype(jnp.int32)
        pos_ref[pl.ds(c * RBLK, RBLK), :] = (
            pos_ref[pl.ds(c * RBLK, RBLK), :] + add)
        return carry

    lax.fori_loop(0, RNB, pass2, 0)

    starts = (lax.broadcasted_iota(jnp.int32, (NB, 1), 0) * BLK
              ).astype(jnp.float32)
    ends = starts + float(BLK - 1)
    fb = jnp.sum((cum_incl <= starts).astype(jnp.float32),
                 axis=1, keepdims=True)
    lb = jnp.sum((cum_incl <= ends).astype(jnp.float32),
                 axis=1, keepdims=True)
    npairs = lb - fb + 1.0

    idn_r = lax.broadcasted_iota(jnp.int32, (NB, NB), 0)
    idn_c = lax.broadcasted_iota(jnp.int32, (NB, NB), 1)
    idn = (idn_r == idn_c).astype(jnp.float32)
    ltn = (idn_r < idn_c).astype(jnp.float32)
    np_row = _dg(npairs, idn, 0, 0)
    ps_row = _dg(np_row, ltn, 1, 0)
    ps_col = _dg(idn, ps_row, 1, 1)
    total = ps_row[0, NB - 1] + np_row[0, NB - 1]

    s_f = lax.broadcasted_iota(jnp.int32, (SM, 1), 0).astype(jnp.float32)
    b_col = jnp.sum((ps_row <= s_f).astype(jnp.float32), axis=1,
                    keepdims=True) - 1.0
    b_col = jnp.clip(b_col, 0.0, float(NB - 1))
    bi_row = lax.broadcasted_iota(jnp.int32, (1, NB), 1).astype(jnp.float32)
    oh_b = (b_col == bi_row).astype(jnp.float32)
    ps_of_b = _dg(oh_b, ps_col, 1, 0)
    fb_of_b = _dg(oh_b, fb, 1, 0)
    lb_of_b = _dg(oh_b, lb, 1, 0)
    e_col = jnp.minimum(fb_of_b + (s_f - ps_of_b), lb_of_b)
    e_col = jnp.clip(e_col, 0.0, float(E - 1))
    valid = (s_f < total).astype(jnp.float32)

    idE = (lt8_r == lt8_c).astype(jnp.float32)
    off_col = _dg(idE, off_excl, 1, 1)
    pad = jnp.zeros((SM - E - 1, 1), jnp.float32)
    offs = jnp.concatenate(
        [off_col, jnp.full((1, 1), float(T), jnp.float32), pad], axis=0)

    meta = jnp.concatenate([b_col, e_col, valid, offs], axis=1)
    meta_ref[...] = meta.astype(jnp.int32)


def _route(ids):
    return pl.pallas_call(
        _routing_body,
        out_shape=(jax.ShapeDtypeStruct((T, 1), jnp.int32),
                   jax.ShapeDtypeStruct((SM, 4), jnp.int32)),
    )(ids)


@functools.lru_cache(maxsize=None)
def _sc_kernels():
    mesh = plsc.VectorSubcoreMesh(core_axis_name="c", subcore_axis_name="s")
    scratch = [pltpu.VMEM((CHUNK,), jnp.int32),
               pltpu.VMEM((CHUNK, D), jnp.float32),
               pltpu.SemaphoreType.DMA]

    @functools.partial(
        pl.kernel,
        out_type=jax.ShapeDtypeStruct((T, D), jnp.float32),
        mesh=mesh, scratch_types=scratch)
    def dispatch(x_hbm, pos_hbm, out_hbm, idx_v, rows_v, sem):
        wid = lax.axis_index("s") * 2 + lax.axis_index("c")
        base = wid * CHUNK
        pltpu.sync_copy(pos_hbm.at[pl.ds(base, CHUNK)], idx_v)
        pltpu.sync_copy(x_hbm.at[pl.ds(base, CHUNK)], rows_v)
        pltpu.async_copy(rows_v, out_hbm.at[idx_v], sem).wait()

    @functools.partial(
        pl.kernel,
        out_type=jax.ShapeDtypeStruct((T, D), jnp.float32),
        mesh=mesh, scratch_types=scratch)
    def combine(y_hbm, pos_hbm, out_hbm, idx_v, rows_v, sem):
        wid = lax.axis_index("s") * 2 + lax.axis_index("c")
        base = wid * CHUNK
        pltpu.sync_copy(pos_hbm.at[pl.ds(base, CHUNK)], idx_v)
        pltpu.async_copy(y_hbm.at[idx_v], rows_v, sem).wait()
        pltpu.sync_copy(rows_v, out_hbm.at[pl.ds(base, CHUNK)])

    return dispatch, combine


def _mlp_body(rb_ref, ex_ref, va_ref, off_ref, x_ref, gw_ref, uw_ref, dw_ref,
              o_ref):
    s = pl.program_id(0)
    e = ex_ref[s]
    base = rb_ref[s] * BLK
    rows = base + lax.broadcasted_iota(jnp.int32, (BLK, 1), 0)
    mask = (rows >= off_ref[e]) & (rows < off_ref[e + 1]) & (va_ref[s] > 0)

    x = x_ref[...]
    g = _dg(x, gw_ref[0], 1, 1)
    u = _dg(x, uw_ref[0], 1, 1)
    h = (g * jax.nn.sigmoid(g)) * u
    y = _dg(h, dw_ref[0], 1, 1)

    prev_rb = rb_ref[jnp.maximum(s - 1, 0)]
    first = (s == 0) | (rb_ref[s] != prev_rb)
    prev = jnp.where(first, jnp.zeros_like(y), o_ref[...])
    o_ref[...] = prev + jnp.where(mask, y, 0.0)


def _mlp(rb, ex, va, off, sorted_x, gate_w, up_w, down_w):
    grid_spec = pltpu.PrefetchScalarGridSpec(
        num_scalar_prefetch=4,
        grid=(S,),
        in_specs=[
            pl.BlockSpec((BLK, D), lambda s, rb, ex, va, off: (rb[s], 0)),
            pl.BlockSpec((1, FF, D), lambda s, rb, ex, va, off: (ex[s], 0, 0)),
            pl.BlockSpec((1, FF, D), lambda s, rb, ex, va, off: (ex[s], 0, 0)),
            pl.BlockSpec((1, D, FF), lambda s, rb, ex, va, off: (ex[s], 0, 0)),
        ],
        out_specs=pl.BlockSpec((BLK, D), lambda s, rb, ex, va, off: (rb[s], 0)),
    )
    return pl.pallas_call(
        _mlp_body,
        grid_spec=grid_spec,
        out_shape=jax.ShapeDtypeStruct((T, D), jnp.float32),
    )(rb, ex, va, off, sorted_x, gate_w, up_w, down_w)


def _scale_body(y_ref, w_ref, o_ref):
    o_ref[...] = y_ref[...] * w_ref[...]


def _scale(y, w):
    return pl.pallas_call(
        _scale_body,
        grid=(NB,),
        in_specs=[pl.BlockSpec((BLK, D), lambda i: (i, 0)),
                  pl.BlockSpec((BLK, 1), lambda i: (i, 0))],
        out_specs=pl.BlockSpec((BLK, D), lambda i: (i, 0)),
        out_shape=jax.ShapeDtypeStruct((T, D), jnp.float32),
    )(y, w)


def kernel(x, topk_ids, topk_weight, gate_w, up_w, down_w):
    ids = topk_ids.reshape(T, 1).astype(jnp.int32)
    pos2d, meta = _route(ids)
    rb = meta[:S, 0]
    ex = meta[:S, 1]
    va = meta[:S, 2]
    off = meta[:E + 1, 3]
    pos = pos2d.reshape(T)

    dispatch, combine = _sc_kernels()
    sorted_x = dispatch(x, pos)
    sorted_y = _mlp(rb, ex, va, off, sorted_x, gate_w, up_w, down_w)
    y_tok = combine(sorted_y, pos)
    return _scale(y_tok, topk_weight.reshape(T, 1).astype(jnp.float32))


# speedup vs baseline: 2.7744x; 1.0442x over previous
import functools

import jax
import jax.numpy as jnp
from jax import lax
from jax.experimental import pallas as pl
from jax.experimental.pallas import tpu as pltpu
from jax.experimental.pallas import tpu_sc as plsc

E = 8
D = 1024
FF = 2048
T = 2048
BLK = 256
NB = T // BLK
SP = NB + E - 1
NBP = NB + E
TP = NBP * BLK
RBLK = 128
RNB = T // RBLK
SM = 32
NWORKERS = 32
CHUNK = T // NWORKERS


def _dg(a, b, ca, cb):
    return lax.dot_general(a, b, (((ca,), (cb,)), ((), ())),
                           preferred_element_type=jnp.float32)


def _routing_body(ids_ref, pos_ref, meta_ref):
    e_row = lax.broadcasted_iota(jnp.int32, (1, E), 1)
    tri_r = lax.broadcasted_iota(jnp.int32, (RBLK, RBLK), 0)
    tri_c = lax.broadcasted_iota(jnp.int32, (RBLK, RBLK), 1)
    tri = (tri_r >= tri_c).astype(jnp.float32)

    def pass1(c, running):
        ids_c = ids_ref[pl.ds(c * RBLK, RBLK), :]
        oh = (ids_c == e_row).astype(jnp.float32)
        ranks = _dg(tri, oh, 1, 0) + running
        grank = jnp.sum(oh * (ranks - 1.0), axis=1, keepdims=True)
        pos_ref[pl.ds(c * RBLK, RBLK), :] = grank.astype(jnp.int32)
        return ranks[RBLK - 1:RBLK, :]

    counts = lax.fori_loop(0, RNB, pass1, jnp.zeros((1, E), jnp.float32))

    lt8_r = lax.broadcasted_iota(jnp.int32, (E, E), 0)
    lt8_c = lax.broadcasted_iota(jnp.int32, (E, E), 1)
    lt8 = (lt8_r < lt8_c).astype(jnp.float32)
    nblk = jnp.floor((counts + float(BLK - 1)) * (1.0 / BLK))
    ps = _dg(nblk, lt8, 1, 0)
    pad_off = ps * float(BLK)

    def pass2(c, carry):
        ids_c = ids_ref[pl.ds(c * RBLK, RBLK), :]
        oh = (ids_c == e_row).astype(jnp.float32)
        add = jnp.sum(oh * pad_off, axis=1, keepdims=True).astype(jnp.int32)
        pos_ref[pl.ds(c * RBLK, RBLK), :] = (
            pos_ref[pl.ds(c * RBLK, RBLK), :] + add)
        return carry

    lax.fori_loop(0, RNB, pass2, 0)

    total = ps[0, E - 1] + nblk[0, E - 1]
    s_f = lax.broadcasted_iota(jnp.int32, (SM, 1), 0).astype(jnp.float32)
    e_col = jnp.sum((ps <= s_f).astype(jnp.float32), axis=1,
                    keepdims=True) - 1.0
    e_col = jnp.clip(e_col, 0.0, float(E - 1))
    valid = (s_f < total).astype(jnp.float32)
    xb = jnp.minimum(s_f, total - 1.0)

    meta = jnp.concatenate([xb, e_col, valid, jnp.zeros((SM, 1))], axis=1)
    meta_ref[...] = meta.astype(jnp.int32)


def _route(ids):
    return pl.pallas_call(
        _routing_body,
        out_shape=(jax.ShapeDtypeStruct((T, 1), jnp.int32),
                   jax.ShapeDtypeStruct((SM, 4), jnp.int32)),
    )(ids)


@functools.lru_cache(maxsize=None)
def _sc_kernels():
    mesh = plsc.VectorSubcoreMesh(core_axis_name="c", subcore_axis_name="s")
    scratch = [pltpu.VMEM((CHUNK,), jnp.int32),
               pltpu.VMEM((CHUNK, D), jnp.float32),
               pltpu.SemaphoreType.DMA]

    @functools.partial(
        pl.kernel,
        out_type=jax.ShapeDtypeStruct((TP, D), jnp.float32),
        mesh=mesh, scratch_types=scratch)
    def dispatch(x_hbm, pos_hbm, out_hbm, idx_v, rows_v, sem):
        wid = lax.axis_index("s") * 2 + lax.axis_index("c")
        base = wid * CHUNK
        pltpu.sync_copy(pos_hbm.at[pl.ds(base, CHUNK)], idx_v)
        pltpu.sync_copy(x_hbm.at[pl.ds(base, CHUNK)], rows_v)
        pltpu.async_copy(rows_v, out_hbm.at[idx_v], sem).wait()

    @functools.partial(
        pl.kernel,
        out_type=jax.ShapeDtypeStruct((T, D), jnp.float32),
        mesh=mesh, scratch_types=scratch)
    def combine(y_hbm, pos_hbm, out_hbm, idx_v, rows_v, sem):
        wid = lax.axis_index("s") * 2 + lax.axis_index("c")
        base = wid * CHUNK
        pltpu.sync_copy(pos_hbm.at[pl.ds(base, CHUNK)], idx_v)
        pltpu.async_copy(y_hbm.at[idx_v], rows_v, sem).wait()
        pltpu.sync_copy(rows_v, out_hbm.at[pl.ds(base, CHUNK)])

    return dispatch, combine


def _mlp_body(xb_ref, ex_ref, va_ref, x_ref, gw_ref, uw_ref, dw_ref, o_ref):
    s = pl.program_id(0)

    @pl.when(va_ref[s] > 0)
    def _():
        x = x_ref[...]
        g = _dg(x, gw_ref[0], 1, 1)
        u = _dg(x, uw_ref[0], 1, 1)
        h = (g * jax.nn.sigmoid(g)) * u
        o_ref[...] = _dg(h, dw_ref[0], 1, 1)


def _mlp(xb, ex, va, sorted_x, gate_w, up_w, down_w):
    grid_spec = pltpu.PrefetchScalarGridSpec(
        num_scalar_prefetch=3,
        grid=(SP,),
        in_specs=[
            pl.BlockSpec((BLK, D), lambda s, xb, ex, va: (xb[s], 0)),
            pl.BlockSpec((1, FF, D), lambda s, xb, ex, va: (ex[s], 0, 0)),
            pl.BlockSpec((1, FF, D), lambda s, xb, ex, va: (ex[s], 0, 0)),
            pl.BlockSpec((1, D, FF), lambda s, xb, ex, va: (ex[s], 0, 0)),
        ],
        out_specs=pl.BlockSpec((BLK, D), lambda s, xb, ex, va: (xb[s], 0)),
    )
    return pl.pallas_call(
        _mlp_body,
        grid_spec=grid_spec,
        out_shape=jax.ShapeDtypeStruct((TP, D), jnp.float32),
    )(xb, ex, va, sorted_x, gate_w, up_w, down_w)


def _scale_body(y_ref, w_ref, o_ref):
    o_ref[...] = y_ref[...] * w_ref[...]


def _scale(y, w):
    return pl.pallas_call(
        _scale_body,
        grid=(NB,),
        in_specs=[pl.BlockSpec((BLK, D), lambda i: (i, 0)),
                  pl.BlockSpec((BLK, 1), lambda i: (i, 0))],
        out_specs=pl.BlockSpec((BLK, D), lambda i: (i, 0)),
        out_shape=jax.ShapeDtypeStruct((T, D), jnp.float32),
    )(y, w)


def kernel(x, topk_ids, topk_weight, gate_w, up_w, down_w):
    ids = topk_ids.reshape(T, 1).astype(jnp.int32)
    pos2d, meta = _route(ids)
    xb = meta[:SP, 0]
    ex = meta[:SP, 1]
    va = meta[:SP, 2]
    pos = pos2d.reshape(T)

    dispatch, combine = _sc_kernels()
    sorted_x = dispatch(x, pos)
    sorted_y = _mlp(xb, ex, va, sorted_x, gate_w, up_w, down_w)
    y_tok = combine(sorted_y, pos)
    return _scale(y_tok, topk_weight.reshape(T, 1).astype(jnp.float32))
